# baseline (speedup 1.0000x reference)
import functools

import jax
import jax.numpy as jnp
from jax.experimental import pallas as pl
from jax.experimental.pallas import tpu as pltpu

EPS = 1e-5


def _stats_cast_kernel(x_ref, w_ref, xb_ref, sum_ref, sq_ref, acc_s, acc_q):
    j = pl.program_id(1)
    nj = pl.num_programs(1)

    @pl.when(j == 0)
    def _():
        acc_s[...] = jnp.zeros_like(acc_s)
        acc_q[...] = jnp.zeros_like(acc_q)

    xb = x_ref[0].astype(jnp.bfloat16)
    xb_ref[0] = xb
    y = jax.lax.dot_general(
        w_ref[...], xb,
        dimension_numbers=(((1,), (0,)), ((), ())),
        preferred_element_type=jnp.float32)
    acc_s[...] += jnp.sum(y, axis=1, keepdims=True)
    acc_q[...] += jnp.sum(y * y, axis=1, keepdims=True)

    @pl.when(j == nj - 1)
    def _():
        sum_ref[0] = acc_s[...]
        sq_ref[0] = acc_q[...]


def _apply_kernel(xb_ref, w_ref, sum_ref, sq_ref, g_ref, beta_ref, o_ref,
                  *, inv_n):
    s = sum_ref[0] + sum_ref[1]
    q = sq_ref[0] + sq_ref[1]
    mean = s * inv_n
    var = q * inv_n - mean * mean
    scale = g_ref[...] * jax.lax.rsqrt(var + EPS)
    shift = beta_ref[...] - mean * scale
    y = jax.lax.dot_general(
        w_ref[...], xb_ref[0],
        dimension_numbers=(((1,), (0,)), ((), ())),
        preferred_element_type=jnp.float32)
    o_ref[0] = jnp.maximum(y * scale + shift, 0.0)


def kernel(x, w, b, g, beta):
    del b
    B, Cin, L = x.shape
    Cout = w.shape[0]

    wb = w.astype(jnp.bfloat16)
    g2 = g.reshape(Cout, 1).astype(jnp.float32)
    beta2 = beta.reshape(Cout, 1).astype(jnp.float32)
    inv_n = 1.0 / float(B * L)

    bh = B // 2

    xb, sums, sqs = pl.pallas_call(
        _stats_cast_kernel,
        out_shape=(jax.ShapeDtypeStruct((B, Cin, L), jnp.bfloat16),
                   jax.ShapeDtypeStruct((2, Cout, 1), jnp.float32),
                   jax.ShapeDtypeStruct((2, Cout, 1), jnp.float32)),
        grid=(2, bh),
        in_specs=[
            pl.BlockSpec((1, Cin, L), lambda i, j: (i * bh + j, 0, 0)),
            pl.BlockSpec((Cout, Cin), lambda i, j: (0, 0)),
        ],
        out_specs=(pl.BlockSpec((1, Cin, L), lambda i, j: (i * bh + j, 0, 0)),
                   pl.BlockSpec((1, Cout, 1), lambda i, j: (i, 0, 0)),
                   pl.BlockSpec((1, Cout, 1), lambda i, j: (i, 0, 0))),
        scratch_shapes=[pltpu.VMEM((Cout, 1), jnp.float32),
                        pltpu.VMEM((Cout, 1), jnp.float32)],
        compiler_params=pltpu.CompilerParams(
            dimension_semantics=("parallel", "arbitrary"),
            vmem_limit_bytes=56 * 1024 * 1024),
    )(x, wb)

    out = pl.pallas_call(
        functools.partial(_apply_kernel, inv_n=inv_n),
        out_shape=jax.ShapeDtypeStruct((B, Cout, L), jnp.float32),
        grid=(B,),
        in_specs=[
            pl.BlockSpec((1, Cin, L), lambda i: (i, 0, 0)),
            pl.BlockSpec((Cout, Cin), lambda i: (0, 0)),
            pl.BlockSpec((2, Cout, 1), lambda i: (0, 0, 0)),
            pl.BlockSpec((2, Cout, 1), lambda i: (0, 0, 0)),
            pl.BlockSpec((Cout, 1), lambda i: (0, 0)),
            pl.BlockSpec((Cout, 1), lambda i: (0, 0)),
        ],
        out_specs=pl.BlockSpec((1, Cout, L), lambda i: (i, 0, 0)),
        compiler_params=pltpu.CompilerParams(
            dimension_semantics=("parallel",),
            vmem_limit_bytes=56 * 1024 * 1024),
    )(xb, wb, sums, sqs, g2, beta2)

    return out


# speedup vs baseline: 1.3263x; 1.3263x over previous
import functools

import jax
import jax.numpy as jnp
from jax.experimental import pallas as pl
from jax.experimental.pallas import tpu as pltpu

EPS = 1e-5


def _fused_kernel(x_ref, w_ref, g_ref, beta_ref, o_ref,
                  xb_ref, acc_x1, acc_g, scale_ref, shift_ref, *, n1, inv_n):
    t = pl.program_id(0)

    @pl.when(t == 0)
    def _():
        acc_x1[...] = jnp.zeros_like(acc_x1)
        acc_g[...] = jnp.zeros_like(acc_g)

    @pl.when(t < n1)
    def _():
        xf = x_ref[0]
        xb = xf.astype(jnp.bfloat16)
        xb_ref[t] = xb
        acc_x1[...] += jnp.sum(xf, axis=1, keepdims=True)
        acc_g[...] += jax.lax.dot_general(
            xb, xb,
            dimension_numbers=(((1,), (1,)), ((), ())),
            preferred_element_type=jnp.float32)

    @pl.when(t == n1)
    def _():
        w32 = w_ref[...].astype(jnp.float32)
        s = jax.lax.dot_general(
            w32, acc_x1[...],
            dimension_numbers=(((1,), (0,)), ((), ())),
            precision=jax.lax.Precision.HIGHEST,
            preferred_element_type=jnp.float32)
        wg = jax.lax.dot_general(
            w32, acc_g[...],
            dimension_numbers=(((1,), (0,)), ((), ())),
            precision=jax.lax.Precision.HIGHEST,
            preferred_element_type=jnp.float32)
        q = jnp.sum(wg * w32, axis=1, keepdims=True)
        mean = s * inv_n
        var = q * inv_n - mean * mean
        scale = g_ref[...] * jax.lax.rsqrt(var + EPS)
        scale_ref[...] = scale
        shift_ref[...] = beta_ref[...] - mean * scale

    @pl.when(t >= n1)
    def _():
        y = jax.lax.dot_general(
            w_ref[...], xb_ref[t - n1],
            dimension_numbers=(((1,), (0,)), ((), ())),
            preferred_element_type=jnp.float32)
        o_ref[0] = jnp.maximum(y * scale_ref[...] + shift_ref[...], 0.0)


def _pick_tile(L):
    for tl in (4096, 2048, 1024, 512, 256, 128):
        if L % tl == 0:
            return tl
    return L


def kernel(x, w, b, g, beta):
    del b
    B, Cin, L = x.shape
    Cout = w.shape[0]

    wb = w.astype(jnp.bfloat16)
    g2 = g.reshape(Cout, 1).astype(jnp.float32)
    beta2 = beta.reshape(Cout, 1).astype(jnp.float32)
    inv_n = 1.0 / float(B * L)

    tl = _pick_tile(L)
    njl = L // tl
    n1 = B * njl

    def x_index(t):
        s = jnp.minimum(t, n1 - 1)
        return (s // njl, 0, s % njl)

    def o_index(t):
        u = jnp.maximum(t - n1, 0)
        return (u // njl, 0, u % njl)

    return pl.pallas_call(
        functools.partial(_fused_kernel, n1=n1, inv_n=inv_n),
        out_shape=jax.ShapeDtypeStruct((B, Cout, L), jnp.float32),
        grid=(2 * n1,),
        in_specs=[
            pl.BlockSpec((1, Cin, tl), x_index),
            pl.BlockSpec((Cout, Cin), lambda t: (0, 0)),
            pl.BlockSpec((Cout, 1), lambda t: (0, 0)),
            pl.BlockSpec((Cout, 1), lambda t: (0, 0)),
        ],
        out_specs=pl.BlockSpec((1, Cout, tl), o_index),
        scratch_shapes=[
            pltpu.VMEM((n1, Cin, tl), jnp.bfloat16),
            pltpu.VMEM((Cin, 1), jnp.float32),
            pltpu.VMEM((Cin, Cin), jnp.float32),
            pltpu.VMEM((Cout, 1), jnp.float32),
            pltpu.VMEM((Cout, 1), jnp.float32),
        ],
        compiler_params=pltpu.CompilerParams(
            dimension_semantics=("arbitrary",),
            vmem_limit_bytes=56 * 1024 * 1024),
    )(x, wb, g2, beta2)
